# Initial kernel scaffold; baseline (speedup 1.0000x reference)
import jax
import jax.numpy as jnp
from jax.experimental import pallas as pl


def kernel(pred_logits, pred_ctrl_points, pred_text_logits, tgt_ctrl_points, tgt_texts):
    raise NotImplementedError("write your pallas kernel here")


# speedup vs baseline: 4.6144x; 4.6144x over previous
import jax
import jax.numpy as jnp
from jax.experimental import pallas as pl
from jax.experimental.pallas import tpu as pltpu

_BS = 2
_NQ = 1000
_NCTRL = 25
_T = 25
_VOC = 96
_NTGT = 20
_CLASS_W = 2.0
_COORD_W = 5.0
_TEXT_W = 2.0
_ALPHA = 0.25
_GAMMA = 2.0
_MATCH = 5
_NEG = -1e9
_U = _T


def _ctc_kernel(xT_ref, tgt_ref, out_ref, lse_s, lpb_s):
    p = pl.program_id(0)

    @pl.when(p % _NTGT == 0)
    def _():
        def lse_body(t, _):
            x_t = xT_ref[0, t]
            m = jnp.max(x_t, axis=0, keepdims=True)
            lse = m + jnp.log(jnp.sum(jnp.exp(x_t - m), axis=0, keepdims=True))
            lse_s[pl.ds(t, 1), :] = lse
            lpb_s[pl.ds(t, 1), :] = x_t[_VOC : _VOC + 1, :] - lse
            return 0

        jax.lax.fori_loop(0, _T, lse_body, 0)

    tgt = tgt_ref[0]
    iota_v = jax.lax.broadcasted_iota(jnp.int32, (_U, _VOC + 1), 1)
    E = (tgt == iota_v).astype(jnp.float32)
    prev = jnp.concatenate([jnp.full((1, 1), -1, jnp.int32), tgt[:-1]], axis=0)
    allow = tgt != prev

    def lp_char(t):
        x_t = xT_ref[0, t]
        lse = lse_s[pl.ds(t, 1), :]
        return jnp.dot(E, x_t, preferred_element_type=jnp.float32) - lse

    neg_row = jnp.full((1, _NQ), _NEG, dtype=jnp.float32)
    row_c = jax.lax.broadcasted_iota(jnp.int32, (_U, 1), 0)
    row_b = jax.lax.broadcasted_iota(jnp.int32, (_U + 1, 1), 0)

    lpc0 = lp_char(0)
    lpb0 = lpb_s[0:1, :]
    alpha_c = jnp.where(row_c == 0, lpc0, _NEG)
    alpha_b = jnp.where(row_b == 0, jnp.broadcast_to(lpb0, (_U + 1, _NQ)), _NEG)

    def step(t, carry):
        alpha_c, alpha_b = carry
        lpc_t = lp_char(t)
        lpb_t = lpb_s[pl.ds(t, 1), :]
        c_shift = jnp.concatenate([neg_row, alpha_c[:-1]], axis=0)
        cs26 = jnp.concatenate([neg_row, alpha_c], axis=0)
        m_b = jnp.maximum(alpha_b, cs26)
        new_b = lpb_t + m_b + jnp.log(jnp.exp(alpha_b - m_b) + jnp.exp(cs26 - m_b))
        a2 = alpha_b[:-1]
        a3 = jnp.where(allow, c_shift, _NEG)
        m_c = jnp.maximum(jnp.maximum(alpha_c, a2), a3)
        s_c = jnp.exp(alpha_c - m_c) + jnp.exp(a2 - m_c) + jnp.exp(a3 - m_c)
        new_c = lpc_t + m_c + jnp.log(s_c)
        return (new_c, new_b)

    alpha_c, alpha_b = jax.lax.fori_loop(1, _T, step, (alpha_c, alpha_b))
    ll = jnp.logaddexp(alpha_c[_U - 1 : _U, :], alpha_b[_U : _U + 1, :])
    out_ref[0] = -ll / float(_U)


def _ctc_call(xT, tgtT):
    return pl.pallas_call(
        _ctc_kernel,
        grid=(_BS * _NTGT,),
        in_specs=[
            pl.BlockSpec((1, _T, _VOC + 1, _NQ), lambda p: (p // _NTGT, 0, 0, 0)),
            pl.BlockSpec((1, _U, 1), lambda p: (p, 0, 0)),
        ],
        out_specs=pl.BlockSpec((1, 1, _NQ), lambda p: (p, 0, 0)),
        out_shape=jax.ShapeDtypeStruct((_BS * _NTGT, 1, _NQ), jnp.float32),
        scratch_shapes=[
            pltpu.VMEM((_T, _NQ), jnp.float32),
            pltpu.VMEM((_T, _NQ), jnp.float32),
        ],
    )(xT, tgtT)


_QB = 200


def _cc_kernel(lg_ref, pts_ref, tp_ref, out_ref):
    p = jax.nn.sigmoid(lg_ref[...])
    pos = _ALPHA * (1.0 - p) * (1.0 - p) * (-jnp.log(p + 1e-8))
    neg = (1.0 - _ALPHA) * p * p * (-jnp.log(1.0 - p + 1e-8))
    cc = jnp.mean(pos - neg, axis=1, keepdims=True)

    pts = pts_ref[...]
    col = jax.lax.broadcasted_iota(jnp.int32, (_QB, _BS * _NTGT), 1)
    acc = jnp.zeros((_QB, _BS * _NTGT), jnp.float32)
    for j in range(_BS * _NTGT):
        d = jnp.sum(jnp.abs(pts - tp_ref[j : j + 1, :]), axis=1, keepdims=True)
        acc = jnp.where(col == j, d, acc)
    out_ref[...] = _CLASS_W * cc + _COORD_W * acc


def _cc_call(lg2, pts2, tpts):
    nblk = (_BS * _NQ) // _QB
    return pl.pallas_call(
        _cc_kernel,
        grid=(nblk,),
        in_specs=[
            pl.BlockSpec((_QB, _NCTRL), lambda i: (i, 0)),
            pl.BlockSpec((_QB, 2 * _NCTRL), lambda i: (i, 0)),
            pl.BlockSpec((_BS * _NTGT, 2 * _NCTRL), lambda i: (0, 0)),
        ],
        out_specs=pl.BlockSpec((_QB, _BS * _NTGT), lambda i: (i, 0)),
        out_shape=jax.ShapeDtypeStruct((_BS * _NQ, _BS * _NTGT), jnp.float32),
    )(lg2, pts2, tpts)


def _match_kernel(ct_ref, tx_ref, cost_ref, idx_ref):
    c = ct_ref[0] + _TEXT_W * tx_ref[0]
    cost_ref[0] = c
    iq = jax.lax.broadcasted_iota(jnp.int32, (_NTGT, _NQ), 1)
    big = jnp.int32(1 << 30)
    for k in range(_MATCH):
        mn = jnp.min(c, axis=1, keepdims=True)
        idx = jnp.min(jnp.where(c == mn, iq, big), axis=1, keepdims=True)
        idx_ref[0, :, k : k + 1] = idx
        c = jnp.where(iq == idx, jnp.float32(3e38), c)


def _match_call(ctT, txT):
    return pl.pallas_call(
        _match_kernel,
        grid=(_BS,),
        in_specs=[
            pl.BlockSpec((1, _NTGT, _NQ), lambda b: (b, 0, 0)),
            pl.BlockSpec((1, _NTGT, _NQ), lambda b: (b, 0, 0)),
        ],
        out_specs=[
            pl.BlockSpec((1, _NTGT, _NQ), lambda b: (b, 0, 0)),
            pl.BlockSpec((1, _NTGT, _MATCH), lambda b: (b, 0, 0)),
        ],
        out_shape=[
            jax.ShapeDtypeStruct((_BS, _NTGT, _NQ), jnp.float32),
            jax.ShapeDtypeStruct((_BS, _NTGT, _MATCH), jnp.int32),
        ],
    )(ctT, txT)


def kernel(pred_logits, pred_ctrl_points, pred_text_logits, tgt_ctrl_points, tgt_texts):
    xT = pred_text_logits.transpose(0, 2, 3, 1)
    tgtT = tgt_texts.astype(jnp.int32).reshape(_BS * _NTGT, _U, 1)
    lg2 = pred_logits.reshape(_BS * _NQ, _NCTRL)
    pts2 = pred_ctrl_points.reshape(_BS * _NQ, 2 * _NCTRL)
    tpts = tgt_ctrl_points.reshape(_BS * _NTGT, 2 * _NCTRL)

    text = _ctc_call(xT, tgtT)
    C2 = _cc_call(lg2, pts2, tpts)

    C = C2.reshape(_BS, _NQ, _BS * _NTGT)
    ctT = jnp.stack(
        [C[b, :, b * _NTGT : (b + 1) * _NTGT].T for b in range(_BS)]
    )
    txT = text.reshape(_BS, _NTGT, _NQ)

    costT, idx = _match_call(ctT, txT)
    cost = costT.transpose(0, 2, 1)
    src = idx.reshape(_BS, _NTGT * _MATCH)
    tgt_idx = jnp.broadcast_to(
        jnp.repeat(jnp.arange(_NTGT, dtype=jnp.int32), _MATCH), (_BS, _NTGT * _MATCH)
    )
    return (C, cost, src, tgt_idx)


# speedup vs baseline: 6.2351x; 1.3512x over previous
import jax
import jax.numpy as jnp
from jax.experimental import pallas as pl
from jax.experimental.pallas import tpu as pltpu

_BS = 2
_NQ = 1000
_NCTRL = 25
_T = 25
_VOC = 96
_NTGT = 20
_CLASS_W = 2.0
_COORD_W = 5.0
_TEXT_W = 2.0
_ALPHA = 0.25
_GAMMA = 2.0
_MATCH = 5
_NEG = -1e9
_U = _T


_UP = _U + 1
_R = _NTGT * _UP


def _ctc_kernel(xT_ref, tgt_ref, out_ref, lse_s, lpb_s):
    def lse_body(t, _):
        x_t = xT_ref[0, t]
        m = jnp.max(x_t, axis=0, keepdims=True)
        lse = m + jnp.log(jnp.sum(jnp.exp(x_t - m), axis=0, keepdims=True))
        lse_s[pl.ds(t, 1), :] = lse
        lpb_s[pl.ds(t, 1), :] = x_t[_VOC : _VOC + 1, :] - lse
        return 0

    jax.lax.fori_loop(0, _T, lse_body, 0)

    tgt = tgt_ref[0]
    iota_v = jax.lax.broadcasted_iota(jnp.int32, (_R, _VOC + 1), 1)
    E = (tgt == iota_v).astype(jnp.float32)
    prev = jnp.concatenate([jnp.full((1, 1), -2, jnp.int32), tgt[:-1]], axis=0)
    allow = tgt != prev
    r_iota = jax.lax.broadcasted_iota(jnp.int32, (_R, 1), 0)
    is_u0 = (r_iota % _UP) == 0

    def lp_char(t):
        x_t = xT_ref[0, t]
        lse = lse_s[pl.ds(t, 1), :]
        return jnp.dot(E, x_t, preferred_element_type=jnp.float32) - lse

    neg = jnp.float32(_NEG)
    neg_row = jnp.full((1, _NQ), _NEG, dtype=jnp.float32)

    lpc0 = lp_char(0)
    lpb0 = lpb_s[0:1, :]
    alpha_c = jnp.where(is_u0, lpc0, neg)
    alpha_b = jnp.where(is_u0, jnp.broadcast_to(lpb0, (_R, _NQ)), neg)

    def step(t, carry):
        ac, ab = carry
        lpc_t = lp_char(t)
        lpb_t = lpb_s[pl.ds(t, 1), :]
        c_sh = jnp.concatenate([neg_row, ac[:-1]], axis=0)
        c_sh = jnp.where(is_u0, neg, c_sh)
        m_b = jnp.maximum(ab, c_sh)
        new_b = lpb_t + m_b + jnp.log(jnp.exp(ab - m_b) + jnp.exp(c_sh - m_b))
        a3 = jnp.where(allow, c_sh, neg)
        m_c = jnp.maximum(jnp.maximum(ac, ab), a3)
        s_c = jnp.exp(ac - m_c) + jnp.exp(ab - m_c) + jnp.exp(a3 - m_c)
        new_c = lpc_t + m_c + jnp.log(s_c)
        return (new_c, new_b)

    ac, ab = jax.lax.fori_loop(1, _T, step, (alpha_c, alpha_b))
    b_sh = jnp.concatenate([ab[1:], neg_row], axis=0)
    out_ref[0] = jnp.logaddexp(ac, b_sh) * jnp.float32(-1.0 / _U)


def _ctc_call(xT, tgt_stack):
    return pl.pallas_call(
        _ctc_kernel,
        grid=(_BS,),
        in_specs=[
            pl.BlockSpec((1, _T, _VOC + 1, _NQ), lambda b: (b, 0, 0, 0)),
            pl.BlockSpec((1, _R, 1), lambda b: (b, 0, 0)),
        ],
        out_specs=pl.BlockSpec((1, _R, _NQ), lambda b: (b, 0, 0)),
        out_shape=jax.ShapeDtypeStruct((_BS, _R, _NQ), jnp.float32),
        scratch_shapes=[
            pltpu.VMEM((_T, _NQ), jnp.float32),
            pltpu.VMEM((_T, _NQ), jnp.float32),
        ],
        compiler_params=pltpu.CompilerParams(
            dimension_semantics=("parallel",),
        ),
    )(xT, tgt_stack)


_QB = 200


def _cc_kernel(lg_ref, pts_ref, tp_ref, out_ref):
    p = jax.nn.sigmoid(lg_ref[...])
    pos = _ALPHA * (1.0 - p) * (1.0 - p) * (-jnp.log(p + 1e-8))
    neg = (1.0 - _ALPHA) * p * p * (-jnp.log(1.0 - p + 1e-8))
    cc = jnp.mean(pos - neg, axis=1, keepdims=True)

    pts = pts_ref[...]
    col = jax.lax.broadcasted_iota(jnp.int32, (_QB, _BS * _NTGT), 1)
    acc = jnp.zeros((_QB, _BS * _NTGT), jnp.float32)
    for j in range(_BS * _NTGT):
        d = jnp.sum(jnp.abs(pts - tp_ref[j : j + 1, :]), axis=1, keepdims=True)
        acc = jnp.where(col == j, d, acc)
    out_ref[...] = _CLASS_W * cc + _COORD_W * acc


def _cc_call(lg2, pts2, tpts):
    nblk = (_BS * _NQ) // _QB
    return pl.pallas_call(
        _cc_kernel,
        grid=(nblk,),
        in_specs=[
            pl.BlockSpec((_QB, _NCTRL), lambda i: (i, 0)),
            pl.BlockSpec((_QB, 2 * _NCTRL), lambda i: (i, 0)),
            pl.BlockSpec((_BS * _NTGT, 2 * _NCTRL), lambda i: (0, 0)),
        ],
        out_specs=pl.BlockSpec((_QB, _BS * _NTGT), lambda i: (i, 0)),
        out_shape=jax.ShapeDtypeStruct((_BS * _NQ, _BS * _NTGT), jnp.float32),
    )(lg2, pts2, tpts)


def _match_kernel(ct_ref, tx_ref, cost_ref, idx_ref):
    c = ct_ref[0] + _TEXT_W * tx_ref[0]
    cost_ref[0] = c
    iq = jax.lax.broadcasted_iota(jnp.int32, (_NTGT, _NQ), 1)
    big = jnp.int32(1 << 30)
    for k in range(_MATCH):
        mn = jnp.min(c, axis=1, keepdims=True)
        idx = jnp.min(jnp.where(c == mn, iq, big), axis=1, keepdims=True)
        idx_ref[0, :, k : k + 1] = idx
        c = jnp.where(iq == idx, jnp.float32(3e38), c)


def _match_call(ctT, txT):
    return pl.pallas_call(
        _match_kernel,
        grid=(_BS,),
        in_specs=[
            pl.BlockSpec((1, _NTGT, _NQ), lambda b: (b, 0, 0)),
            pl.BlockSpec((1, _NTGT, _NQ), lambda b: (b, 0, 0)),
        ],
        out_specs=[
            pl.BlockSpec((1, _NTGT, _NQ), lambda b: (b, 0, 0)),
            pl.BlockSpec((1, _NTGT, _MATCH), lambda b: (b, 0, 0)),
        ],
        out_shape=[
            jax.ShapeDtypeStruct((_BS, _NTGT, _NQ), jnp.float32),
            jax.ShapeDtypeStruct((_BS, _NTGT, _MATCH), jnp.int32),
        ],
    )(ctT, txT)


def kernel(pred_logits, pred_ctrl_points, pred_text_logits, tgt_ctrl_points, tgt_texts):
    xT = pred_text_logits.transpose(0, 2, 3, 1)
    tgt_stack = jnp.pad(
        tgt_texts.astype(jnp.int32).reshape(_BS, _NTGT, _U),
        ((0, 0), (0, 0), (0, 1)),
        constant_values=-1,
    ).reshape(_BS, _R, 1)
    lg2 = pred_logits.reshape(_BS * _NQ, _NCTRL)
    pts2 = pred_ctrl_points.reshape(_BS * _NQ, 2 * _NCTRL)
    tpts = tgt_ctrl_points.reshape(_BS * _NTGT, 2 * _NCTRL)

    text_full = _ctc_call(xT, tgt_stack)
    text = text_full[:, _U - 1 :: _UP, :]
    C2 = _cc_call(lg2, pts2, tpts)

    C = C2.reshape(_BS, _NQ, _BS * _NTGT)
    ctT = jnp.stack(
        [C[b, :, b * _NTGT : (b + 1) * _NTGT].T for b in range(_BS)]
    )
    costT, idx = _match_call(ctT, text)
    cost = costT.transpose(0, 2, 1)
    src = idx.reshape(_BS, _NTGT * _MATCH)
    tgt_idx = jnp.broadcast_to(
        jnp.repeat(jnp.arange(_NTGT, dtype=jnp.int32), _MATCH), (_BS, _NTGT * _MATCH)
    )
    return (C, cost, src, tgt_idx)
